# baseline (speedup 1.0000x reference)
import jax
import jax.numpy as jnp
from jax import lax
from jax.experimental import pallas as pl
from jax.experimental.pallas import tpu as pltpu
from jax.experimental.pallas import tpu_sc as plsc

B = 4096
F = 26
D = 64
BF = B * F
NC, NS = 2, 16
NW = NC * NS
PER_W = BF // NW
CHUNK = 32
NCH = PER_W // CHUNK
NROUNDS = NCH // 2
LANES = 16
FIELD_SIZE = 100000


def _body(x_hbm, w_hbm, out_hbm, idx_v, tv, buf0, buf1, ob0, ob1,
          gs0, gs1, cs0, cs1):
    wid = lax.axis_index("s") * NC + lax.axis_index("c")
    base = wid * PER_W
    pltpu.sync_copy(x_hbm.at[pl.ds(base, PER_W)], idx_v)

    def off(t, carry):
        pos = base + t * LANES + lax.iota(jnp.int32, LANES)
        sl = pl.ds(t * LANES, LANES)
        v = idx_v[sl] + lax.rem(pos, F) * FIELD_SIZE
        idx_v[sl] = v
        tv[sl] = v & ~7
        return carry

    lax.fori_loop(0, PER_W // LANES, off, 0)

    bufs = (buf0, buf1)
    obufs = (ob0, ob1)
    gsems = (gs0, gs1)
    csems = (cs0, cs1)

    def fire_chunk(j, b):
        for g in range(CHUNK // LANES):
            vec = tv[pl.ds(j * CHUNK + g * LANES, LANES)]
            for l in range(LANES):
                r8 = pl.multiple_of(vec[l], 8)
                pltpu.async_copy(
                    w_hbm.at[pl.ds(r8, 8)],
                    bufs[b].at[g * LANES + l], gsems[b])

    def gather_drain(b):
        for _ in range(CHUNK):
            pltpu.make_async_copy(
                w_hbm.at[pl.ds(0, 8)], bufs[b].at[0], gsems[b]).wait()

    def copy_desc(j, b):
        return pltpu.make_async_copy(
            obufs[b],
            out_hbm.at[pl.ds(wid * (PER_W // 2) + j * (CHUNK // 2),
                             CHUNK // 2)],
            csems[b])

    def select(j, b):
        for g in range(CHUNK // LANES):
            hv = idx_v[pl.ds(j * CHUNK + g * LANES, LANES)]
            for l in range(LANES):
                rit = hv[l] & 7
                srow = g * LANES + l
                orow = g * (LANES // 2) + (l >> 1)
                ocol = (l & 1) * D
                for t in range(D // LANES):
                    obufs[b][orow, pl.ds(ocol + t * LANES, LANES)] = (
                        bufs[b][srow, rit, pl.ds(t * LANES, LANES)])

    fire_chunk(0, 0)
    fire_chunk(1, 1)

    def rnd(k, carry):
        for b in range(2):
            j = 2 * k + b
            gather_drain(b)

            @pl.when(k > 0)
            def _():
                copy_desc(j - 2, b).wait()

            select(j, b)
            copy_desc(j, b).start()

            @pl.when(k < NROUNDS - 1)
            def _():
                fire_chunk(j + 2, b)

        return carry

    lax.fori_loop(0, NROUNDS, rnd, 0)
    copy_desc(NCH - 2, 0).wait()
    copy_desc(NCH - 1, 1).wait()


def kernel(x, weight):
    mesh = plsc.VectorSubcoreMesh(
        core_axis_name="c", subcore_axis_name="s",
        num_cores=NC, num_subcores=NS,
    )
    lookup = pl.kernel(
        _body,
        out_type=jax.ShapeDtypeStruct((BF // 2, 2 * D), jnp.float32),
        mesh=mesh,
        scratch_types=[
            pltpu.VMEM((PER_W,), jnp.int32),
            pltpu.VMEM((PER_W,), jnp.int32),
            pltpu.VMEM((CHUNK, 8, D), jnp.float32),
            pltpu.VMEM((CHUNK, 8, D), jnp.float32),
            pltpu.VMEM((CHUNK // 2, 2 * D), jnp.float32),
            pltpu.VMEM((CHUNK // 2, 2 * D), jnp.float32),
            pltpu.SemaphoreType.DMA,
            pltpu.SemaphoreType.DMA,
            pltpu.SemaphoreType.DMA,
            pltpu.SemaphoreType.DMA,
        ],
    )
    out = lookup(x.reshape(BF), weight)
    return out.reshape(B, F * D)
